# Initial kernel scaffold; baseline (speedup 1.0000x reference)
import jax
import jax.numpy as jnp
from jax.experimental import pallas as pl


def kernel(x, f, w, W_msg, b_msg, W_upd, b_upd, edge_index):
    raise NotImplementedError("write your pallas kernel here")


# speedup vs baseline: 5.1646x; 5.1646x over previous
import functools

import jax
import jax.numpy as jnp
from jax import lax
from jax.experimental import pallas as pl
from jax.experimental.pallas import tpu as pltpu
from jax.experimental.pallas import tpu_sc as plsc

N = 10000
E = 320000
D = 128
DE = 16
PAY = 144
NC = 2
NS = 16
NW = NC * NS
C = 128
CH_PER_TILE = -(-E // (NW * C))
E_PAD = NW * CH_PER_TILE * C
N_ACC = 10240
RPT = N_ACC // NS
BN = 1024


def _sc_body(fn_hbm, w_hbm, src_hbm, dst_hbm, zpay_hbm, zde_hbm,
             accf_hbm, accw_hbm,
             srcv, dstv, rows, wrows, shf, shw, sem):
    c = lax.axis_index("c")
    s = lax.axis_index("s")
    wid = s * NC + c
    r0 = s * RPT
    pltpu.sync_copy(zpay_hbm.at[pl.ds(r0, RPT)], shf.at[pl.ds(r0, RPT)])
    pltpu.sync_copy(zde_hbm.at[pl.ds(r0, RPT)], shw.at[pl.ds(r0, RPT)])
    plsc.subcore_barrier()

    base = wid * (CH_PER_TILE * C)

    def chunk(i, carry):
        off = pl.multiple_of(base + i * C, C)
        pltpu.sync_copy(src_hbm.at[pl.ds(off, C)], srcv)
        pltpu.sync_copy(dst_hbm.at[pl.ds(off, C)], dstv)
        pltpu.async_copy(fn_hbm.at[srcv], rows, sem).wait()
        pltpu.sync_copy(w_hbm.at[pl.ds(off, C)], wrows)
        pltpu.sync_copy(rows, shf.at[dstv], add=True)
        pltpu.sync_copy(wrows, shw.at[dstv], add=True)
        return carry

    lax.fori_loop(0, CH_PER_TILE, chunk, 0)
    plsc.subcore_barrier()
    pltpu.sync_copy(shf.at[pl.ds(r0, RPT)], accf_hbm.at[c, pl.ds(r0, RPT)])
    pltpu.sync_copy(shw.at[pl.ds(r0, RPT)], accw_hbm.at[c, pl.ds(r0, RPT)])


_sc_segsum = functools.partial(
    pl.kernel,
    out_type=[
        jax.ShapeDtypeStruct((NC, N_ACC, PAY), jnp.float32),
        jax.ShapeDtypeStruct((NC, N_ACC, DE), jnp.float32),
    ],
    mesh=plsc.VectorSubcoreMesh(core_axis_name="c", subcore_axis_name="s"),
    scratch_types=[
        pltpu.VMEM((C,), jnp.int32),
        pltpu.VMEM((C,), jnp.int32),
        pltpu.VMEM((C, PAY), jnp.float32),
        pltpu.VMEM((C, DE), jnp.float32),
        pltpu.VMEM_SHARED((N_ACC, PAY), jnp.float32),
        pltpu.VMEM_SHARED((N_ACC, DE), jnp.float32),
        pltpu.SemaphoreType.DMA,
    ],
    compiler_params=pltpu.CompilerParams(use_tc_tiling_on_sc=False),
)(_sc_body)


def _prep_body(x_ref, f_ref, o_ref):
    x = x_ref[...]
    q = jnp.sum(x * x, axis=1, keepdims=True)
    one = jnp.ones_like(q)
    pad = jnp.zeros((x.shape[0], PAY - D - 5), jnp.float32)
    o_ref[...] = jnp.concatenate([f_ref[...], x, q, one, pad], axis=1)


def _epi_body(acc_ref, accw_ref, fn_ref, wg_ref, w2_ref, w3_ref, w4_ref,
              wu_ref, bu_ref, o_ref):
    A = acc_ref[0] + acc_ref[1]
    Bw = accw_ref[0] + accw_ref[1]
    fn = fn_ref[...]
    f = fn[:, :D]
    xv = fn[:, D:D + 3]
    q = fn[:, D + 3:D + 4]
    Ax = A[:, D:D + 3]
    Aq = A[:, D + 3:D + 4]
    deg = A[:, D + 4:D + 5]
    s = Aq + deg * q - 2.0 * jnp.sum(xv * Ax, axis=1, keepdims=True)
    m = (jnp.dot(A, wg_ref[...], preferred_element_type=jnp.float32)
         + jnp.dot(Bw, w3_ref[...], preferred_element_type=jnp.float32)
         + deg * jnp.dot(f, w2_ref[...], preferred_element_type=jnp.float32)
         + s * w4_ref[...])
    o_ref[...] = (jnp.dot(m + f, wu_ref[...], preferred_element_type=jnp.float32)
                  + bu_ref[...])


def kernel(x, f, w, W_msg, b_msg, W_upd, b_upd, edge_index):
    x = x.astype(jnp.float32)
    f = f.astype(jnp.float32)
    w = w.astype(jnp.float32)

    x_pad = jnp.zeros((N_ACC, 3), jnp.float32).at[:N].set(x)
    f_pad = jnp.zeros((N_ACC, D), jnp.float32).at[:N].set(f)
    src = jnp.full((E_PAD,), N, jnp.int32).at[:E].set(edge_index[0])
    dst = jnp.full((E_PAD,), N, jnp.int32).at[:E].set(edge_index[1])
    w_pad = jnp.zeros((E_PAD, DE), jnp.float32).at[:E].set(w)
    zpay = jnp.zeros((N_ACC, PAY), jnp.float32)
    zde = jnp.zeros((N_ACC, DE), jnp.float32)

    wg = jnp.zeros((PAY, D), jnp.float32)
    wg = wg.at[:D].set(W_msg[:D])
    wg = wg.at[D + 4].set(b_msg)
    w2 = W_msg[D:2 * D]
    w3 = W_msg[2 * D:2 * D + DE]
    w4 = W_msg[2 * D + DE:2 * D + DE + 1]
    bu = b_upd.reshape(1, D)

    fnode = pl.pallas_call(
        _prep_body,
        out_shape=jax.ShapeDtypeStruct((N_ACC, PAY), jnp.float32),
        grid=(N_ACC // BN,),
        in_specs=[
            pl.BlockSpec((BN, 3), lambda i: (i, 0)),
            pl.BlockSpec((BN, D), lambda i: (i, 0)),
        ],
        out_specs=pl.BlockSpec((BN, PAY), lambda i: (i, 0)),
    )(x_pad, f_pad)

    accf, accw = _sc_segsum(fnode, w_pad, src, dst, zpay, zde)

    out = pl.pallas_call(
        _epi_body,
        out_shape=jax.ShapeDtypeStruct((N_ACC, D), jnp.float32),
        grid=(N_ACC // BN,),
        in_specs=[
            pl.BlockSpec((NC, BN, PAY), lambda i: (0, i, 0)),
            pl.BlockSpec((NC, BN, DE), lambda i: (0, i, 0)),
            pl.BlockSpec((BN, PAY), lambda i: (i, 0)),
            pl.BlockSpec((PAY, D), lambda i: (0, 0)),
            pl.BlockSpec((D, D), lambda i: (0, 0)),
            pl.BlockSpec((DE, D), lambda i: (0, 0)),
            pl.BlockSpec((1, D), lambda i: (0, 0)),
            pl.BlockSpec((D, D), lambda i: (0, 0)),
            pl.BlockSpec((1, D), lambda i: (0, 0)),
        ],
        out_specs=pl.BlockSpec((BN, D), lambda i: (i, 0)),
    )(accf, accw, fnode, wg, w2, w3, w4, W_upd, bu)

    return out[:N]


# speedup vs baseline: 5.2371x; 1.0140x over previous
import functools

import jax
import jax.numpy as jnp
from jax import lax
from jax.experimental import pallas as pl
from jax.experimental.pallas import tpu as pltpu
from jax.experimental.pallas import tpu_sc as plsc

N = 10000
E = 320000
D = 128
DE = 16
PAY = 144
NC = 2
NS = 16
NW = NC * NS
C = 64
EROWS = E // C
NCH = 160
ROWS_PAD = NW * NCH
N_ACC = 10240
RPT = N_ACC // NS
BN = 1024
BP = 1000


def _sc_body(fn_hbm, w_hbm, src_hbm, dst_hbm,
             accf_hbm, accw_hbm,
             srcv0, srcv1, srcv2, srcv3,
             dstv0, dstv1, dstv2, dstv3,
             wr0, wr1, wr2, wr3,
             rows0, rows1, shf, shw,
             ls0, ls1, ls2, ls3,
             ts0, ts1, ts2, ts3,
             gs0, gs1, ss0, ss1):
    c = lax.axis_index("c")
    s = lax.axis_index("s")
    wid = s * NC + c
    r0 = s * RPT
    row0 = wid * NCH

    srcv = (srcv0, srcv1, srcv2, srcv3)
    dstv = (dstv0, dstv1, dstv2, dstv3)
    wr = (wr0, wr1, wr2, wr3)
    rows = (rows0, rows1)
    lsem = (ls0, ls1, ls2, ls3)
    tsem = (ts0, ts1, ts2, ts3)
    gsem = (gs0, gs1)
    ssem = (ss0, ss1)

    zf32 = jnp.zeros((16,), jnp.float32)

    def zrow(i, carry):
        for j in range(PAY // 16):
            rows0[i, pl.ds(j * 16, 16)] = zf32
        wr0[i, pl.ds(0, 16)] = zf32
        return carry

    lax.fori_loop(0, C, zrow, 0)
    for k in range(RPT // C):
        pltpu.async_copy(rows0, shf.at[pl.ds(r0 + k * C, C)], gs0)
        pltpu.async_copy(wr0, shw.at[pl.ds(r0 + k * C, C)], ss0)
    for k in range(RPT // C):
        pltpu.make_async_copy(rows0, shf.at[pl.ds(0, C)], gs0).wait()
        pltpu.make_async_copy(wr0, shw.at[pl.ds(0, C)], ss0).wait()
    plsc.subcore_barrier()

    def issue_l(i, k):
        woff = pl.multiple_of(jnp.minimum(row0 + i, EROWS - 1) * C, C)
        ioff = pl.multiple_of((row0 + i) * C, C)
        pltpu.async_copy(src_hbm.at[pl.ds(ioff, C)], srcv[k], lsem[k])
        pltpu.async_copy(dst_hbm.at[pl.ds(ioff, C)], dstv[k], lsem[k])
        pltpu.async_copy(w_hbm.at[pl.ds(woff, C)], wr[k], lsem[k])

    def wait_l(k):
        pltpu.make_async_copy(src_hbm.at[pl.ds(0, C)], srcv[k], lsem[k]).wait()
        pltpu.make_async_copy(dst_hbm.at[pl.ds(0, C)], dstv[k], lsem[k]).wait()
        pltpu.make_async_copy(w_hbm.at[pl.ds(0, C)], wr[k], lsem[k]).wait()

    def issue_g(k, b):
        pltpu.async_copy(fn_hbm.at[srcv[k]], rows[b], gsem[b])

    def wait_g(k, b):
        pltpu.make_async_copy(fn_hbm.at[srcv[k]], rows[b], gsem[b]).wait()

    def issue_s(k, b):
        pltpu.async_copy(rows[b], shf.at[dstv[k]], ssem[b], add=True)
        pltpu.async_copy(wr[k], shw.at[dstv[k]], tsem[k], add=True)

    def wait_s(k, b):
        pltpu.make_async_copy(rows[b], shf.at[dstv[k]], ssem[b]).wait()
        pltpu.make_async_copy(wr[k], shw.at[dstv[k]], tsem[k]).wait()

    issue_l(0, 0)
    issue_l(1, 1)

    def block(j, carry):
        for k in range(4):
            i = 4 * j + k
            b = k % 2
            wait_l(k)
            @pl.when(i >= 2)
            def _():
                wait_s((k + 2) % 4, b)

            @pl.when(i + 2 < NCH)
            def _():
                issue_l(i + 2, (k + 2) % 4)

            issue_g(k, b)
            wait_g(k, b)
            issue_s(k, b)
        return carry

    lax.fori_loop(0, NCH // 4, block, 0)
    wait_s(2, 0)
    wait_s(3, 1)
    plsc.subcore_barrier()
    pltpu.sync_copy(shf.at[pl.ds(r0, RPT)], accf_hbm.at[c, pl.ds(r0, RPT)])
    pltpu.sync_copy(shw.at[pl.ds(r0, RPT)], accw_hbm.at[c, pl.ds(r0, RPT)])


_sc_segsum = functools.partial(
    pl.kernel,
    out_type=[
        jax.ShapeDtypeStruct((NC, N_ACC, PAY), jnp.float32),
        jax.ShapeDtypeStruct((NC, N_ACC, DE), jnp.float32),
    ],
    mesh=plsc.VectorSubcoreMesh(core_axis_name="c", subcore_axis_name="s"),
    scratch_types=[
        pltpu.VMEM((C,), jnp.int32),
        pltpu.VMEM((C,), jnp.int32),
        pltpu.VMEM((C,), jnp.int32),
        pltpu.VMEM((C,), jnp.int32),
        pltpu.VMEM((C,), jnp.int32),
        pltpu.VMEM((C,), jnp.int32),
        pltpu.VMEM((C,), jnp.int32),
        pltpu.VMEM((C,), jnp.int32),
        pltpu.VMEM((C, DE), jnp.float32),
        pltpu.VMEM((C, DE), jnp.float32),
        pltpu.VMEM((C, DE), jnp.float32),
        pltpu.VMEM((C, DE), jnp.float32),
        pltpu.VMEM((C, PAY), jnp.float32),
        pltpu.VMEM((C, PAY), jnp.float32),
        pltpu.VMEM_SHARED((N_ACC, PAY), jnp.float32),
        pltpu.VMEM_SHARED((N_ACC, DE), jnp.float32),
        pltpu.SemaphoreType.DMA,
        pltpu.SemaphoreType.DMA,
        pltpu.SemaphoreType.DMA,
        pltpu.SemaphoreType.DMA,
        pltpu.SemaphoreType.DMA,
        pltpu.SemaphoreType.DMA,
        pltpu.SemaphoreType.DMA,
        pltpu.SemaphoreType.DMA,
        pltpu.SemaphoreType.DMA,
        pltpu.SemaphoreType.DMA,
        pltpu.SemaphoreType.DMA,
        pltpu.SemaphoreType.DMA,
    ],
    compiler_params=pltpu.CompilerParams(use_tc_tiling_on_sc=False),
)(_sc_body)


def _prep_body(x_ref, f_ref, o_ref):
    x = x_ref[...]
    q = jnp.sum(x * x, axis=1, keepdims=True)
    one = jnp.ones_like(q)
    pad = jnp.zeros((x.shape[0], PAY - D - 5), jnp.float32)
    o_ref[...] = jnp.concatenate([f_ref[...], x, q, one, pad], axis=1)


def _epi_body(acc_ref, accw_ref, fn_ref, wg_ref, w2_ref, w3_ref, w4_ref,
              wu_ref, bu_ref, o_ref):
    A = acc_ref[0] + acc_ref[1]
    Bw = accw_ref[0] + accw_ref[1]
    fn = fn_ref[...]
    f = fn[:, :D]
    xv = fn[:, D:D + 3]
    q = fn[:, D + 3:D + 4]
    Ax = A[:, D:D + 3]
    Aq = A[:, D + 3:D + 4]
    deg = A[:, D + 4:D + 5]
    s = Aq + deg * q - 2.0 * jnp.sum(xv * Ax, axis=1, keepdims=True)
    m = (jnp.dot(A, wg_ref[...], preferred_element_type=jnp.float32)
         + jnp.dot(Bw, w3_ref[...], preferred_element_type=jnp.float32)
         + deg * jnp.dot(f, w2_ref[...], preferred_element_type=jnp.float32)
         + s * w4_ref[...])
    o_ref[...] = (jnp.dot(m + f, wu_ref[...], preferred_element_type=jnp.float32)
                  + bu_ref[...])


def kernel(x, f, w, W_msg, b_msg, W_upd, b_upd, edge_index):
    x = x.astype(jnp.float32)
    f = f.astype(jnp.float32)
    w = w.astype(jnp.float32)

    src1 = jnp.zeros((ROWS_PAD * C,), jnp.int32).at[:E].set(edge_index[0])
    dst1 = jnp.full((ROWS_PAD * C,), N, jnp.int32).at[:E].set(edge_index[1])

    wg = jnp.zeros((PAY, D), jnp.float32)
    wg = wg.at[:D].set(W_msg[:D])
    wg = wg.at[D + 4].set(b_msg)
    w2 = W_msg[D:2 * D]
    w3 = W_msg[2 * D:2 * D + DE]
    w4 = W_msg[2 * D + DE:2 * D + DE + 1]
    bu = b_upd.reshape(1, D)

    fnode = pl.pallas_call(
        _prep_body,
        out_shape=jax.ShapeDtypeStruct((N_ACC, PAY), jnp.float32),
        grid=(N // BP,),
        in_specs=[
            pl.BlockSpec((BP, 3), lambda i: (i, 0)),
            pl.BlockSpec((BP, D), lambda i: (i, 0)),
        ],
        out_specs=pl.BlockSpec((BP, PAY), lambda i: (i, 0)),
    )(x, f)

    accf, accw = _sc_segsum(fnode, w, src1, dst1)

    out = pl.pallas_call(
        _epi_body,
        out_shape=jax.ShapeDtypeStruct((N_ACC, D), jnp.float32),
        grid=(N_ACC // BN,),
        in_specs=[
            pl.BlockSpec((NC, BN, PAY), lambda i: (0, i, 0)),
            pl.BlockSpec((NC, BN, DE), lambda i: (0, i, 0)),
            pl.BlockSpec((BN, PAY), lambda i: (i, 0)),
            pl.BlockSpec((PAY, D), lambda i: (0, 0)),
            pl.BlockSpec((D, D), lambda i: (0, 0)),
            pl.BlockSpec((DE, D), lambda i: (0, 0)),
            pl.BlockSpec((1, D), lambda i: (0, 0)),
            pl.BlockSpec((D, D), lambda i: (0, 0)),
            pl.BlockSpec((1, D), lambda i: (0, 0)),
        ],
        out_specs=pl.BlockSpec((BN, D), lambda i: (i, 0)),
    )(accf, accw, fnode, wg, w2, w3, w4, W_upd, bu)

    return out[:N]


# speedup vs baseline: 6.2963x; 1.2022x over previous
import functools

import jax
import jax.numpy as jnp
from jax import lax
from jax.experimental import pallas as pl
from jax.experimental.pallas import tpu as pltpu
from jax.experimental.pallas import tpu_sc as plsc

N = 10000
E = 320000
D = 128
DE = 16
PAY = 144
NC = 2
NS = 16
NW = NC * NS
C = 80
EROWS = E // C
K0 = 176
K1 = 80
ROWS_PAD = NS * (K0 + K1)
N_ACC = 10240
RPT = N_ACC // NS
BN = 1024
BP = 1000


def _sc_body(fn_hbm, w_hbm, sd_hbm,
             accf_hbm, accw_hbm,
             sdv0, sdv1, sdv2, sdv3,
             rows0, rows1, wr0, wr1, shf, shw,
             ls0, ls1, ls2, ls3,
             gs0, gs1, ss0, ss1, ts0, ts1):
    c = lax.axis_index("c")
    s = lax.axis_index("s")
    r0 = s * RPT
    nch = jnp.where(c == 0, K0, K1)
    row0 = jnp.where(c == 0, s * K0, NS * K0 + s * K1)

    sdv = (sdv0, sdv1, sdv2, sdv3)
    rows = (rows0, rows1)
    wr = (wr0, wr1)
    lsem = (ls0, ls1, ls2, ls3)
    gsem = (gs0, gs1)
    ssem = (ss0, ss1)
    tsem = (ts0, ts1)

    zf32 = jnp.zeros((16,), jnp.float32)

    def zrow(i, carry):
        for j in range(PAY // 16):
            rows0[i, pl.ds(j * 16, 16)] = zf32
        wr0[i, pl.ds(0, 16)] = zf32
        return carry

    lax.fori_loop(0, C, zrow, 0)
    for k in range(RPT // C):
        pltpu.async_copy(rows0, shf.at[pl.ds(r0 + k * C, C)], gs0)
        pltpu.async_copy(wr0, shw.at[pl.ds(r0 + k * C, C)], ss0)
    for k in range(RPT // C):
        pltpu.make_async_copy(rows0, shf.at[pl.ds(0, C)], gs0).wait()
        pltpu.make_async_copy(wr0, shw.at[pl.ds(0, C)], ss0).wait()
    plsc.subcore_barrier()

    def issue_l(i, k):
        soff = pl.multiple_of((row0 + i) * 2, 2)
        pltpu.async_copy(sd_hbm.at[pl.ds(soff, 2)], sdv[k], lsem[k])

    def wait_l(k):
        pltpu.make_async_copy(sd_hbm.at[pl.ds(0, 2)], sdv[k], lsem[k]).wait()

    def issue_g(i, k, b):
        woff = pl.multiple_of(jnp.minimum(row0 + i, EROWS - 1) * C, C)
        pltpu.async_copy(fn_hbm.at[sdv[k].at[0]], rows[b], gsem[b])
        pltpu.async_copy(w_hbm.at[pl.ds(woff, C)], wr[b], gsem[b])

    def wait_g(k, b):
        pltpu.make_async_copy(fn_hbm.at[sdv[k].at[0]], rows[b], gsem[b]).wait()
        pltpu.make_async_copy(w_hbm.at[pl.ds(0, C)], wr[b], gsem[b]).wait()

    def issue_s(k, b):
        pltpu.async_copy(rows[b], shf.at[sdv[k].at[1]], ssem[b], add=True)
        pltpu.async_copy(wr[b], shw.at[sdv[k].at[1]], tsem[b], add=True)

    def wait_s(k, b):
        pltpu.make_async_copy(rows[b], shf.at[sdv[k].at[1]], ssem[b]).wait()
        pltpu.make_async_copy(wr[b], shw.at[sdv[k].at[1]], tsem[b]).wait()

    issue_l(0, 0)
    issue_l(1, 1)

    def block(j, carry):
        for k in range(4):
            i = 4 * j + k
            b = k % 2
            wait_l(k)
            @pl.when(i >= 2)
            def _():
                wait_s((k + 2) % 4, b)

            issue_g(i, k, b)

            @pl.when(i >= 1)
            def _():
                wait_g((k + 3) % 4, 1 - b)
                issue_s((k + 3) % 4, 1 - b)

            @pl.when(i + 2 < nch)
            def _():
                issue_l(i + 2, (k + 2) % 4)
        return carry

    lax.fori_loop(0, nch // 4, block, 0)
    wait_g(3, 1)
    issue_s(3, 1)
    wait_s(2, 0)
    wait_s(3, 1)
    plsc.subcore_barrier()
    pltpu.sync_copy(shf.at[pl.ds(r0, RPT)], accf_hbm.at[c, pl.ds(r0, RPT)])
    pltpu.sync_copy(shw.at[pl.ds(r0, RPT)], accw_hbm.at[c, pl.ds(r0, RPT)])


_sc_segsum = functools.partial(
    pl.kernel,
    out_type=[
        jax.ShapeDtypeStruct((NC, N_ACC, PAY), jnp.float32),
        jax.ShapeDtypeStruct((NC, N_ACC, DE), jnp.float32),
    ],
    mesh=plsc.VectorSubcoreMesh(core_axis_name="c", subcore_axis_name="s"),
    scratch_types=[
        pltpu.VMEM((2, C), jnp.int32),
        pltpu.VMEM((2, C), jnp.int32),
        pltpu.VMEM((2, C), jnp.int32),
        pltpu.VMEM((2, C), jnp.int32),
        pltpu.VMEM((C, PAY), jnp.float32),
        pltpu.VMEM((C, PAY), jnp.float32),
        pltpu.VMEM((C, DE), jnp.float32),
        pltpu.VMEM((C, DE), jnp.float32),
        pltpu.VMEM_SHARED((N_ACC, PAY), jnp.float32),
        pltpu.VMEM_SHARED((N_ACC, DE), jnp.float32),
        pltpu.SemaphoreType.DMA,
        pltpu.SemaphoreType.DMA,
        pltpu.SemaphoreType.DMA,
        pltpu.SemaphoreType.DMA,
        pltpu.SemaphoreType.DMA,
        pltpu.SemaphoreType.DMA,
        pltpu.SemaphoreType.DMA,
        pltpu.SemaphoreType.DMA,
        pltpu.SemaphoreType.DMA,
        pltpu.SemaphoreType.DMA,
    ],
    compiler_params=pltpu.CompilerParams(use_tc_tiling_on_sc=False),
)(_sc_body)


def _prep_body(x_ref, f_ref, o_ref):
    x = x_ref[...]
    q = jnp.sum(x * x, axis=1, keepdims=True)
    one = jnp.ones_like(q)
    pad = jnp.zeros((x.shape[0], PAY - D - 5), jnp.float32)
    o_ref[...] = jnp.concatenate([f_ref[...], x, q, one, pad], axis=1)


def _epi_body(acc_ref, accw_ref, fn_ref, wg_ref, w2_ref, w3_ref, w4_ref,
              wu_ref, bu_ref, o_ref):
    A = acc_ref[0] + acc_ref[1]
    Bw = accw_ref[0] + accw_ref[1]
    fn = fn_ref[...]
    f = fn[:, :D]
    xv = fn[:, D:D + 3]
    q = fn[:, D + 3:D + 4]
    Ax = A[:, D:D + 3]
    Aq = A[:, D + 3:D + 4]
    deg = A[:, D + 4:D + 5]
    s = Aq + deg * q - 2.0 * jnp.sum(xv * Ax, axis=1, keepdims=True)
    m = (jnp.dot(A, wg_ref[...], preferred_element_type=jnp.float32)
         + jnp.dot(Bw, w3_ref[...], preferred_element_type=jnp.float32)
         + deg * jnp.dot(f, w2_ref[...], preferred_element_type=jnp.float32)
         + s * w4_ref[...])
    o_ref[...] = (jnp.dot(m + f, wu_ref[...], preferred_element_type=jnp.float32)
                  + bu_ref[...])


def kernel(x, f, w, W_msg, b_msg, W_upd, b_upd, edge_index):
    x = x.astype(jnp.float32)
    f = f.astype(jnp.float32)
    w = w.astype(jnp.float32)

    srcm = jnp.zeros((ROWS_PAD, C), jnp.int32).at[:EROWS].set(
        edge_index[0].reshape(EROWS, C))
    dstm = jnp.full((ROWS_PAD, C), N, jnp.int32).at[:EROWS].set(
        edge_index[1].reshape(EROWS, C))
    sd = jnp.stack([srcm, dstm], axis=1).reshape(2 * ROWS_PAD, C)

    wg = jnp.zeros((PAY, D), jnp.float32)
    wg = wg.at[:D].set(W_msg[:D])
    wg = wg.at[D + 4].set(b_msg)
    w2 = W_msg[D:2 * D]
    w3 = W_msg[2 * D:2 * D + DE]
    w4 = W_msg[2 * D + DE:2 * D + DE + 1]
    bu = b_upd.reshape(1, D)

    fnode = pl.pallas_call(
        _prep_body,
        out_shape=jax.ShapeDtypeStruct((N_ACC, PAY), jnp.float32),
        grid=(N // BP,),
        in_specs=[
            pl.BlockSpec((BP, 3), lambda i: (i, 0)),
            pl.BlockSpec((BP, D), lambda i: (i, 0)),
        ],
        out_specs=pl.BlockSpec((BP, PAY), lambda i: (i, 0)),
    )(x, f)

    accf, accw = _sc_segsum(fnode, w, sd)

    out = pl.pallas_call(
        _epi_body,
        out_shape=jax.ShapeDtypeStruct((N_ACC, D), jnp.float32),
        grid=(N_ACC // BN,),
        in_specs=[
            pl.BlockSpec((NC, BN, PAY), lambda i: (0, i, 0)),
            pl.BlockSpec((NC, BN, DE), lambda i: (0, i, 0)),
            pl.BlockSpec((BN, PAY), lambda i: (i, 0)),
            pl.BlockSpec((PAY, D), lambda i: (0, 0)),
            pl.BlockSpec((D, D), lambda i: (0, 0)),
            pl.BlockSpec((DE, D), lambda i: (0, 0)),
            pl.BlockSpec((1, D), lambda i: (0, 0)),
            pl.BlockSpec((D, D), lambda i: (0, 0)),
            pl.BlockSpec((1, D), lambda i: (0, 0)),
        ],
        out_specs=pl.BlockSpec((BN, D), lambda i: (i, 0)),
    )(accf, accw, fnode, wg, w2, w3, w4, W_upd, bu)

    return out[:N]


# speedup vs baseline: 11.6608x; 1.8520x over previous
import functools

import jax
import jax.numpy as jnp
from jax import lax
from jax.experimental import pallas as pl
from jax.experimental.pallas import tpu as pltpu
from jax.experimental.pallas import tpu_sc as plsc

N = 10000
E = 320000
D = 128
DE = 16
PAY = 144
NC = 2
NS = 16
NW = NC * NS
C = 80
EROWS = E // C
A0 = 132
EA = 7
B1 = 116
EB = 1
T0 = NS * A0 + 4 * EA
N_ACC = 10240
RPT = N_ACC // NS
BN = 1024
BP = 1000


def _sc_body(fn_hbm, w_hbm, sd_hbm,
             accf_hbm, accw_hbm,
             sdv0, sdv1, sdv2, sdv3,
             rows0, rows1, wr0, wr1, shf, shw,
             ls0, ls1, ls2, ls3,
             gs0, gs1, ss0, ss1, ts0, ts1):
    c = lax.axis_index("c")
    s = lax.axis_index("s")
    r0 = s * RPT
    nch = jnp.where(c == 0,
                    A0 + 4 * (s < EA).astype(jnp.int32),
                    B1 + 4 * (s < EB).astype(jnp.int32))
    row0 = jnp.where(c == 0,
                     s * A0 + 4 * jnp.minimum(s, EA),
                     T0 + s * B1 + 4 * jnp.minimum(s, EB))

    sdv = (sdv0, sdv1, sdv2, sdv3)
    rows = (rows0, rows1)
    wr = (wr0, wr1)
    lsem = (ls0, ls1, ls2, ls3)
    gsem = (gs0, gs1)
    ssem = (ss0, ss1)
    tsem = (ts0, ts1)

    zf32 = jnp.zeros((16,), jnp.float32)

    def zrow(i, carry):
        for j in range(PAY // 16):
            rows0[i, pl.ds(j * 16, 16)] = zf32
        wr0[i, pl.ds(0, 16)] = zf32
        return carry

    lax.fori_loop(0, C, zrow, 0)
    for k in range(RPT // C):
        pltpu.async_copy(rows0, shf.at[pl.ds(r0 + k * C, C)], gs0)
        pltpu.async_copy(wr0, shw.at[pl.ds(r0 + k * C, C)], ss0)
    for k in range(RPT // C):
        pltpu.make_async_copy(rows0, shf.at[pl.ds(0, C)], gs0).wait()
        pltpu.make_async_copy(wr0, shw.at[pl.ds(0, C)], ss0).wait()
    plsc.subcore_barrier()

    def issue_l(i, k):
        soff = pl.multiple_of((row0 + i) * 2, 2)
        pltpu.async_copy(sd_hbm.at[pl.ds(soff, 2)], sdv[k], lsem[k])

    def wait_l(k):
        pltpu.make_async_copy(sd_hbm.at[pl.ds(0, 2)], sdv[k], lsem[k]).wait()

    def issue_g(i, k, b):
        woff = pl.multiple_of((row0 + i) * C, C)
        pltpu.async_copy(fn_hbm.at[sdv[k].at[0]], rows[b], gsem[b])
        pltpu.async_copy(w_hbm.at[pl.ds(woff, C)], wr[b], gsem[b])

    def wait_g(k, b):
        pltpu.make_async_copy(fn_hbm.at[sdv[k].at[0]], rows[b], gsem[b]).wait()
        pltpu.make_async_copy(w_hbm.at[pl.ds(0, C)], wr[b], gsem[b]).wait()

    def issue_s(k, b):
        pltpu.async_copy(rows[b], shf.at[sdv[k].at[1]], ssem[b], add=True)
        pltpu.async_copy(wr[b], shw.at[sdv[k].at[1]], tsem[b], add=True)

    def wait_s(k, b):
        pltpu.make_async_copy(rows[b], shf.at[sdv[k].at[1]], ssem[b]).wait()
        pltpu.make_async_copy(wr[b], shw.at[sdv[k].at[1]], tsem[b]).wait()

    issue_l(0, 0)
    issue_l(1, 1)

    def block(j, carry):
        for k in range(4):
            i = 4 * j + k
            b = k % 2
            wait_l(k)
            @pl.when(i >= 2)
            def _():
                wait_s((k + 2) % 4, b)

            issue_g(i, k, b)

            @pl.when(i >= 1)
            def _():
                wait_g((k + 3) % 4, 1 - b)
                issue_s((k + 3) % 4, 1 - b)

            @pl.when(i + 2 < nch)
            def _():
                issue_l(i + 2, (k + 2) % 4)
        return carry

    lax.fori_loop(0, nch // 4, block, 0)
    wait_g(3, 1)
    issue_s(3, 1)
    wait_s(2, 0)
    wait_s(3, 1)
    plsc.subcore_barrier()
    pltpu.sync_copy(shf.at[pl.ds(r0, RPT)], accf_hbm.at[c, pl.ds(r0, RPT)])
    pltpu.sync_copy(shw.at[pl.ds(r0, RPT)], accw_hbm.at[c, pl.ds(r0, RPT)])


_sc_segsum = functools.partial(
    pl.kernel,
    out_type=[
        jax.ShapeDtypeStruct((NC, N_ACC, PAY), jnp.float32),
        jax.ShapeDtypeStruct((NC, N_ACC, DE), jnp.float32),
    ],
    mesh=plsc.VectorSubcoreMesh(core_axis_name="c", subcore_axis_name="s"),
    scratch_types=[
        pltpu.VMEM((2, C), jnp.int32),
        pltpu.VMEM((2, C), jnp.int32),
        pltpu.VMEM((2, C), jnp.int32),
        pltpu.VMEM((2, C), jnp.int32),
        pltpu.VMEM((C, PAY), jnp.float32),
        pltpu.VMEM((C, PAY), jnp.float32),
        pltpu.VMEM((C, DE), jnp.float32),
        pltpu.VMEM((C, DE), jnp.float32),
        pltpu.VMEM_SHARED((N_ACC, PAY), jnp.float32),
        pltpu.VMEM_SHARED((N_ACC, DE), jnp.float32),
        pltpu.SemaphoreType.DMA,
        pltpu.SemaphoreType.DMA,
        pltpu.SemaphoreType.DMA,
        pltpu.SemaphoreType.DMA,
        pltpu.SemaphoreType.DMA,
        pltpu.SemaphoreType.DMA,
        pltpu.SemaphoreType.DMA,
        pltpu.SemaphoreType.DMA,
        pltpu.SemaphoreType.DMA,
        pltpu.SemaphoreType.DMA,
    ],
    compiler_params=pltpu.CompilerParams(use_tc_tiling_on_sc=False),
)(_sc_body)


def _prep_body(x_ref, f_ref, ei_ref, o_ref, sd_ref):
    x = x_ref[...]
    q = jnp.sum(x * x, axis=1, keepdims=True)
    one = jnp.ones_like(q)
    pad = jnp.zeros((x.shape[0], PAY - D - 5), jnp.float32)
    o_ref[...] = jnp.concatenate([f_ref[...], x, q, one, pad], axis=1)
    e = ei_ref[...]
    sd_ref[...] = jnp.stack([e[0], e[1]], axis=1).reshape(sd_ref.shape)


def _epi_body(acc_ref, accw_ref, fn_ref, wg_ref, w2_ref, w3_ref, w4_ref,
              wu_ref, bu_ref, o_ref):
    A = acc_ref[0] + acc_ref[1]
    Bw = accw_ref[0] + accw_ref[1]
    fn = fn_ref[...]
    f = fn[:, :D]
    xv = fn[:, D:D + 3]
    q = fn[:, D + 3:D + 4]
    Ax = A[:, D:D + 3]
    Aq = A[:, D + 3:D + 4]
    deg = A[:, D + 4:D + 5]
    s = Aq + deg * q - 2.0 * jnp.sum(xv * Ax, axis=1, keepdims=True)
    m = (jnp.dot(A, wg_ref[...], preferred_element_type=jnp.float32)
         + jnp.dot(Bw, w3_ref[...], preferred_element_type=jnp.float32)
         + deg * jnp.dot(f, w2_ref[...], preferred_element_type=jnp.float32)
         + s * w4_ref[...])
    o_ref[...] = (jnp.dot(m + f, wu_ref[...], preferred_element_type=jnp.float32)
                  + bu_ref[...])


def kernel(x, f, w, W_msg, b_msg, W_upd, b_upd, edge_index):
    x = x.astype(jnp.float32)
    f = f.astype(jnp.float32)
    w = w.astype(jnp.float32)

    ei3 = edge_index.reshape(2, EROWS, C)

    wg = jnp.zeros((PAY, D), jnp.float32)
    wg = wg.at[:D].set(W_msg[:D])
    wg = wg.at[D + 4].set(b_msg)
    w2 = W_msg[D:2 * D]
    w3 = W_msg[2 * D:2 * D + DE]
    w4 = W_msg[2 * D + DE:2 * D + DE + 1]
    bu = b_upd.reshape(1, D)

    nblk = N // BP
    erb = EROWS // nblk
    fnode, sd = pl.pallas_call(
        _prep_body,
        out_shape=[
            jax.ShapeDtypeStruct((N_ACC, PAY), jnp.float32),
            jax.ShapeDtypeStruct((2 * EROWS, C), jnp.int32),
        ],
        grid=(nblk,),
        in_specs=[
            pl.BlockSpec((BP, 3), lambda i: (i, 0)),
            pl.BlockSpec((BP, D), lambda i: (i, 0)),
            pl.BlockSpec((2, erb, C), lambda i: (0, i, 0)),
        ],
        out_specs=[
            pl.BlockSpec((BP, PAY), lambda i: (i, 0)),
            pl.BlockSpec((2 * erb, C), lambda i: (i, 0)),
        ],
    )(x, f, ei3)

    accf, accw = _sc_segsum(fnode, w, sd)

    out = pl.pallas_call(
        _epi_body,
        out_shape=jax.ShapeDtypeStruct((N_ACC, D), jnp.float32),
        grid=(N_ACC // BN,),
        in_specs=[
            pl.BlockSpec((NC, BN, PAY), lambda i: (0, i, 0)),
            pl.BlockSpec((NC, BN, DE), lambda i: (0, i, 0)),
            pl.BlockSpec((BN, PAY), lambda i: (i, 0)),
            pl.BlockSpec((PAY, D), lambda i: (0, 0)),
            pl.BlockSpec((D, D), lambda i: (0, 0)),
            pl.BlockSpec((DE, D), lambda i: (0, 0)),
            pl.BlockSpec((1, D), lambda i: (0, 0)),
            pl.BlockSpec((D, D), lambda i: (0, 0)),
            pl.BlockSpec((1, D), lambda i: (0, 0)),
        ],
        out_specs=pl.BlockSpec((BN, D), lambda i: (i, 0)),
    )(accf, accw, fnode, wg, w2, w3, w4, W_upd, bu)

    return out[:N]


# speedup vs baseline: 13.9934x; 1.2000x over previous
import functools

import jax
import jax.numpy as jnp
from jax import lax
from jax.experimental import pallas as pl
from jax.experimental.pallas import tpu as pltpu
from jax.experimental.pallas import tpu_sc as plsc

N = 10000
E = 320000
D = 128
DE = 16
PAY = 144
NC = 2
NS = 16
NW = NC * NS
C = 80
EROWS = E // C
A0 = 132
EA = 7
B1 = 116
EB = 1
T0 = NS * A0 + 4 * EA
N_ACC = 10240
RPT = N_ACC // NS
BN = 1024
BP = 1000


def _sc_body(fn_hbm, sd_hbm,
             accf_hbm,
             sdv0, sdv1, sdv2, sdv3,
             rows0, rows1, shf,
             ls0, ls1, ls2, ls3,
             gs0, gs1, ss0, ss1):
    c = lax.axis_index("c")
    s = lax.axis_index("s")
    r0 = s * RPT
    nch = jnp.where(c == 0,
                    A0 + 4 * (s < EA).astype(jnp.int32),
                    B1 + 4 * (s < EB).astype(jnp.int32))
    row0 = jnp.where(c == 0,
                     s * A0 + 4 * jnp.minimum(s, EA),
                     T0 + s * B1 + 4 * jnp.minimum(s, EB))

    sdv = (sdv0, sdv1, sdv2, sdv3)
    rows = (rows0, rows1)
    lsem = (ls0, ls1, ls2, ls3)
    gsem = (gs0, gs1)
    ssem = (ss0, ss1)

    zf32 = jnp.zeros((16,), jnp.float32)

    def zrow(i, carry):
        for j in range(PAY // 16):
            rows0[i, pl.ds(j * 16, 16)] = zf32
        return carry

    lax.fori_loop(0, C, zrow, 0)
    for k in range(RPT // C):
        pltpu.async_copy(rows0, shf.at[pl.ds(r0 + k * C, C)], gs0)
    for k in range(RPT // C):
        pltpu.make_async_copy(rows0, shf.at[pl.ds(0, C)], gs0).wait()
    plsc.subcore_barrier()

    def issue_l(i, k):
        soff = pl.multiple_of((row0 + i) * 2, 2)
        pltpu.async_copy(sd_hbm.at[pl.ds(soff, 2)], sdv[k], lsem[k])

    def wait_l(k):
        pltpu.make_async_copy(sd_hbm.at[pl.ds(0, 2)], sdv[k], lsem[k]).wait()

    def issue_g(i, k, b):
        pltpu.async_copy(fn_hbm.at[sdv[k].at[0]], rows[b], gsem[b])

    def wait_g(k, b):
        pltpu.make_async_copy(fn_hbm.at[sdv[k].at[0]], rows[b], gsem[b]).wait()

    def issue_s(k, b):
        pltpu.async_copy(rows[b], shf.at[sdv[k].at[1]], ssem[b], add=True)

    def wait_s(k, b):
        pltpu.make_async_copy(rows[b], shf.at[sdv[k].at[1]], ssem[b]).wait()

    issue_l(0, 0)
    issue_l(1, 1)

    def block(j, carry):
        for k in range(4):
            i = 4 * j + k
            b = k % 2
            wait_l(k)
            @pl.when(i >= 2)
            def _():
                wait_s((k + 2) % 4, b)

            issue_g(i, k, b)

            @pl.when(i >= 1)
            def _():
                wait_g((k + 3) % 4, 1 - b)
                issue_s((k + 3) % 4, 1 - b)

            @pl.when(i + 2 < nch)
            def _():
                issue_l(i + 2, (k + 2) % 4)
        return carry

    lax.fori_loop(0, nch // 4, block, 0)
    wait_g(3, 1)
    issue_s(3, 1)
    wait_s(2, 0)
    wait_s(3, 1)
    plsc.subcore_barrier()
    pltpu.sync_copy(shf.at[pl.ds(r0, RPT)], accf_hbm.at[c, pl.ds(r0, RPT)])


_sc_segsum = functools.partial(
    pl.kernel,
    out_type=jax.ShapeDtypeStruct((NC, N_ACC, PAY), jnp.float32),
    mesh=plsc.VectorSubcoreMesh(core_axis_name="c", subcore_axis_name="s"),
    scratch_types=[
        pltpu.VMEM((2, C), jnp.int32),
        pltpu.VMEM((2, C), jnp.int32),
        pltpu.VMEM((2, C), jnp.int32),
        pltpu.VMEM((2, C), jnp.int32),
        pltpu.VMEM((C, PAY), jnp.float32),
        pltpu.VMEM((C, PAY), jnp.float32),
        pltpu.VMEM_SHARED((N_ACC, PAY), jnp.float32),
        pltpu.SemaphoreType.DMA,
        pltpu.SemaphoreType.DMA,
        pltpu.SemaphoreType.DMA,
        pltpu.SemaphoreType.DMA,
        pltpu.SemaphoreType.DMA,
        pltpu.SemaphoreType.DMA,
        pltpu.SemaphoreType.DMA,
        pltpu.SemaphoreType.DMA,
    ],
    compiler_params=pltpu.CompilerParams(use_tc_tiling_on_sc=False),
)(_sc_body)


CW = 128
WROWS = E // CW
KW = 80
WPAD = NW * KW * CW - E


def _scw_body(w_hbm, dst_hbm, accw_hbm,
              dv0, dv1, dv2, dv3, wr0, wr1, shw,
              ls0, ls1, ls2, ls3, gs0, gs1, ss0, ss1):
    c = lax.axis_index("c")
    s = lax.axis_index("s")
    r0 = s * RPT
    row0 = (s * NC + c) * KW

    dv = (dv0, dv1, dv2, dv3)
    wr = (wr0, wr1)
    lsem = (ls0, ls1, ls2, ls3)
    gsem = (gs0, gs1)
    ssem = (ss0, ss1)

    zf32 = jnp.zeros((16,), jnp.float32)

    def zrow(i, carry):
        wr0[i, pl.ds(0, 16)] = zf32
        return carry

    lax.fori_loop(0, CW, zrow, 0)
    for k in range(RPT // CW):
        pltpu.async_copy(wr0, shw.at[pl.ds(r0 + k * CW, CW)], gs0)
    for k in range(RPT // CW):
        pltpu.make_async_copy(wr0, shw.at[pl.ds(0, CW)], gs0).wait()
    plsc.subcore_barrier()

    def issue_l(i, k):
        doff = pl.multiple_of((row0 + i) * CW, CW)
        pltpu.async_copy(dst_hbm.at[pl.ds(doff, CW)], dv[k], lsem[k])

    def wait_l(k):
        pltpu.make_async_copy(dst_hbm.at[pl.ds(0, CW)], dv[k], lsem[k]).wait()

    def issue_g(i, k, b):
        woff = pl.multiple_of(jnp.minimum(row0 + i, WROWS - 1) * CW, CW)
        pltpu.async_copy(w_hbm.at[pl.ds(woff, CW)], wr[b], gsem[b])

    def wait_g(k, b):
        pltpu.make_async_copy(w_hbm.at[pl.ds(0, CW)], wr[b], gsem[b]).wait()

    def issue_s(k, b):
        pltpu.async_copy(wr[b], shw.at[dv[k]], ssem[b], add=True)

    def wait_s(k, b):
        pltpu.make_async_copy(wr[b], shw.at[dv[k]], ssem[b]).wait()

    issue_l(0, 0)
    issue_l(1, 1)

    def block(j, carry):
        for k in range(4):
            i = 4 * j + k
            b = k % 2
            wait_l(k)

            @pl.when(i >= 2)
            def _():
                wait_s((k + 2) % 4, b)

            issue_g(i, k, b)

            @pl.when(i >= 1)
            def _():
                wait_g((k + 3) % 4, 1 - b)
                issue_s((k + 3) % 4, 1 - b)

            @pl.when(i + 2 < KW)
            def _():
                issue_l(i + 2, (k + 2) % 4)
        return carry

    lax.fori_loop(0, KW // 4, block, 0)
    wait_g(3, 1)
    issue_s(3, 1)
    wait_s(2, 0)
    wait_s(3, 1)
    plsc.subcore_barrier()
    pltpu.sync_copy(shw.at[pl.ds(r0, RPT)], accw_hbm.at[c, pl.ds(r0, RPT)])


_scw_segsum = functools.partial(
    pl.kernel,
    out_type=jax.ShapeDtypeStruct((NC, N_ACC, DE), jnp.float32),
    mesh=plsc.VectorSubcoreMesh(core_axis_name="c", subcore_axis_name="s"),
    scratch_types=[
        pltpu.VMEM((CW,), jnp.int32),
        pltpu.VMEM((CW,), jnp.int32),
        pltpu.VMEM((CW,), jnp.int32),
        pltpu.VMEM((CW,), jnp.int32),
        pltpu.VMEM((CW, DE), jnp.float32),
        pltpu.VMEM((CW, DE), jnp.float32),
        pltpu.VMEM_SHARED((N_ACC, DE), jnp.float32),
        pltpu.SemaphoreType.DMA,
        pltpu.SemaphoreType.DMA,
        pltpu.SemaphoreType.DMA,
        pltpu.SemaphoreType.DMA,
        pltpu.SemaphoreType.DMA,
        pltpu.SemaphoreType.DMA,
        pltpu.SemaphoreType.DMA,
        pltpu.SemaphoreType.DMA,
    ],
    compiler_params=pltpu.CompilerParams(use_tc_tiling_on_sc=False),
)(_scw_body)


def _prep_body(x_ref, f_ref, ei_ref, o_ref, sd_ref):
    x = x_ref[...]
    q = jnp.sum(x * x, axis=1, keepdims=True)
    one = jnp.ones_like(q)
    pad = jnp.zeros((x.shape[0], PAY - D - 5), jnp.float32)
    o_ref[...] = jnp.concatenate([f_ref[...], x, q, one, pad], axis=1)
    e = ei_ref[...]
    sd_ref[...] = jnp.stack([e[0], e[1]], axis=1).reshape(sd_ref.shape)


def _epi_body(acc_ref, accw_ref, fn_ref, wg_ref, w2_ref, w3_ref, w4_ref,
              wu_ref, bu_ref, o_ref):
    A = acc_ref[0] + acc_ref[1]
    Bw = accw_ref[0] + accw_ref[1]
    fn = fn_ref[...]
    f = fn[:, :D]
    xv = fn[:, D:D + 3]
    q = fn[:, D + 3:D + 4]
    Ax = A[:, D:D + 3]
    Aq = A[:, D + 3:D + 4]
    deg = A[:, D + 4:D + 5]
    s = Aq + deg * q - 2.0 * jnp.sum(xv * Ax, axis=1, keepdims=True)
    m = (jnp.dot(A, wg_ref[...], preferred_element_type=jnp.float32)
         + jnp.dot(Bw, w3_ref[...], preferred_element_type=jnp.float32)
         + deg * jnp.dot(f, w2_ref[...], preferred_element_type=jnp.float32)
         + s * w4_ref[...])
    o_ref[...] = (jnp.dot(m + f, wu_ref[...], preferred_element_type=jnp.float32)
                  + bu_ref[...])


def kernel(x, f, w, W_msg, b_msg, W_upd, b_upd, edge_index):
    x = x.astype(jnp.float32)
    f = f.astype(jnp.float32)
    w = w.astype(jnp.float32)

    ei3 = edge_index.reshape(2, EROWS, C)
    dstw = jnp.concatenate(
        [edge_index[1],
         N + (jnp.arange(WPAD, dtype=jnp.int32) % (N_ACC - N))])

    wg = jnp.zeros((PAY, D), jnp.float32)
    wg = wg.at[:D].set(W_msg[:D])
    wg = wg.at[D + 4].set(b_msg)
    w2 = W_msg[D:2 * D]
    w3 = W_msg[2 * D:2 * D + DE]
    w4 = W_msg[2 * D + DE:2 * D + DE + 1]
    bu = b_upd.reshape(1, D)

    nblk = N // BP
    erb = EROWS // nblk
    fnode, sd = pl.pallas_call(
        _prep_body,
        out_shape=[
            jax.ShapeDtypeStruct((N_ACC, PAY), jnp.float32),
            jax.ShapeDtypeStruct((2 * EROWS, C), jnp.int32),
        ],
        grid=(nblk,),
        in_specs=[
            pl.BlockSpec((BP, 3), lambda i: (i, 0)),
            pl.BlockSpec((BP, D), lambda i: (i, 0)),
            pl.BlockSpec((2, erb, C), lambda i: (0, i, 0)),
        ],
        out_specs=[
            pl.BlockSpec((BP, PAY), lambda i: (i, 0)),
            pl.BlockSpec((2 * erb, C), lambda i: (i, 0)),
        ],
    )(x, f, ei3)

    accf = _sc_segsum(fnode, sd)
    accw = _scw_segsum(w, dstw)

    out = pl.pallas_call(
        _epi_body,
        out_shape=jax.ShapeDtypeStruct((N_ACC, D), jnp.float32),
        grid=(N_ACC // BN,),
        in_specs=[
            pl.BlockSpec((NC, BN, PAY), lambda i: (0, i, 0)),
            pl.BlockSpec((NC, BN, DE), lambda i: (0, i, 0)),
            pl.BlockSpec((BN, PAY), lambda i: (i, 0)),
            pl.BlockSpec((PAY, D), lambda i: (0, 0)),
            pl.BlockSpec((D, D), lambda i: (0, 0)),
            pl.BlockSpec((DE, D), lambda i: (0, 0)),
            pl.BlockSpec((1, D), lambda i: (0, 0)),
            pl.BlockSpec((D, D), lambda i: (0, 0)),
            pl.BlockSpec((1, D), lambda i: (0, 0)),
        ],
        out_specs=pl.BlockSpec((BN, D), lambda i: (i, 0)),
    )(accf, accw, fnode, wg, w2, w3, w4, W_upd, bu)

    return out[:N]


# speedup vs baseline: 14.4187x; 1.0304x over previous
import functools

import jax
import jax.numpy as jnp
from jax import lax
from jax.experimental import pallas as pl
from jax.experimental.pallas import tpu as pltpu
from jax.experimental.pallas import tpu_sc as plsc

N = 10000
E = 320000
D = 128
DE = 16
PAY = 144
NC = 2
NS = 16
NW = NC * NS
C = 80
EROWS = E // C
A0 = 128
EA = 0
B1 = 120
EB = 8
T0 = NS * A0 + 4 * EA
N_ACC = 10240
RPT = N_ACC // NS
BN = 1000
BP = 1000


def _sc_body(fn_hbm, sd_hbm,
             accf_hbm,
             sdv0, sdv1, sdv2, sdv3,
             rows0, rows1, shf,
             ls0, ls1, ls2, ls3,
             gs0, gs1, ss0, ss1):
    c = lax.axis_index("c")
    s = lax.axis_index("s")
    r0 = s * RPT
    nch = jnp.where(c == 0,
                    A0 + 4 * (s < EA).astype(jnp.int32),
                    B1 + 4 * (s < EB).astype(jnp.int32))
    row0 = jnp.where(c == 0,
                     s * A0 + 4 * jnp.minimum(s, EA),
                     T0 + s * B1 + 4 * jnp.minimum(s, EB))

    sdv = (sdv0, sdv1, sdv2, sdv3)
    rows = (rows0, rows1)
    lsem = (ls0, ls1, ls2, ls3)
    gsem = (gs0, gs1)
    ssem = (ss0, ss1)

    zf32 = jnp.zeros((16,), jnp.float32)

    def zrow(i, carry):
        for j in range(PAY // 16):
            rows0[i, pl.ds(j * 16, 16)] = zf32
        return carry

    lax.fori_loop(0, C, zrow, 0)
    for k in range(RPT // C):
        pltpu.async_copy(rows0, shf.at[pl.ds(r0 + k * C, C)], gs0)
    for k in range(RPT // C):
        pltpu.make_async_copy(rows0, shf.at[pl.ds(0, C)], gs0).wait()
    plsc.subcore_barrier()

    def issue_l(i, k):
        soff = pl.multiple_of((row0 + i) * 2, 2)
        pltpu.async_copy(sd_hbm.at[pl.ds(soff, 2)], sdv[k], lsem[k])

    def wait_l(k):
        pltpu.make_async_copy(sd_hbm.at[pl.ds(0, 2)], sdv[k], lsem[k]).wait()

    def issue_g(i, k, b):
        pltpu.async_copy(fn_hbm.at[sdv[k].at[0]], rows[b], gsem[b])

    def wait_g(k, b):
        pltpu.make_async_copy(fn_hbm.at[sdv[k].at[0]], rows[b], gsem[b]).wait()

    def issue_s(k, b):
        pltpu.async_copy(rows[b], shf.at[sdv[k].at[1]], ssem[b], add=True)

    def wait_s(k, b):
        pltpu.make_async_copy(rows[b], shf.at[sdv[k].at[1]], ssem[b]).wait()

    issue_l(0, 0)
    issue_l(1, 1)

    def block(j, carry):
        for k in range(4):
            i = 4 * j + k
            b = k % 2
            wait_l(k)
            @pl.when(i >= 2)
            def _():
                wait_s((k + 2) % 4, b)

            issue_g(i, k, b)

            @pl.when(i >= 1)
            def _():
                wait_g((k + 3) % 4, 1 - b)
                issue_s((k + 3) % 4, 1 - b)

            @pl.when(i + 2 < nch)
            def _():
                issue_l(i + 2, (k + 2) % 4)
        return carry

    lax.fori_loop(0, nch // 4, block, 0)
    wait_g(3, 1)
    issue_s(3, 1)
    wait_s(2, 0)
    wait_s(3, 1)
    plsc.subcore_barrier()
    pltpu.sync_copy(shf.at[pl.ds(r0, RPT)], accf_hbm.at[c, pl.ds(r0, RPT)])


_sc_segsum = functools.partial(
    pl.kernel,
    out_type=jax.ShapeDtypeStruct((NC, N_ACC, PAY), jnp.float32),
    mesh=plsc.VectorSubcoreMesh(core_axis_name="c", subcore_axis_name="s"),
    scratch_types=[
        pltpu.VMEM((2, C), jnp.int32),
        pltpu.VMEM((2, C), jnp.int32),
        pltpu.VMEM((2, C), jnp.int32),
        pltpu.VMEM((2, C), jnp.int32),
        pltpu.VMEM((C, PAY), jnp.float32),
        pltpu.VMEM((C, PAY), jnp.float32),
        pltpu.VMEM_SHARED((N_ACC, PAY), jnp.float32),
        pltpu.SemaphoreType.DMA,
        pltpu.SemaphoreType.DMA,
        pltpu.SemaphoreType.DMA,
        pltpu.SemaphoreType.DMA,
        pltpu.SemaphoreType.DMA,
        pltpu.SemaphoreType.DMA,
        pltpu.SemaphoreType.DMA,
        pltpu.SemaphoreType.DMA,
    ],
    compiler_params=pltpu.CompilerParams(use_tc_tiling_on_sc=False),
)(_sc_body)


CW = 128
WROWS = E // CW
KW = 80
WPAD = NW * KW * CW - E


def _scw_body(w_hbm, dst_hbm, accw_hbm,
              dv0, dv1, dv2, dv3, wr0, wr1, shw,
              ls0, ls1, ls2, ls3, gs0, gs1, ss0, ss1):
    c = lax.axis_index("c")
    s = lax.axis_index("s")
    r0 = s * RPT
    row0 = (s * NC + c) * KW

    dv = (dv0, dv1, dv2, dv3)
    wr = (wr0, wr1)
    lsem = (ls0, ls1, ls2, ls3)
    gsem = (gs0, gs1)
    ssem = (ss0, ss1)

    zf32 = jnp.zeros((16,), jnp.float32)

    def zrow(i, carry):
        wr0[i, pl.ds(0, 16)] = zf32
        return carry

    lax.fori_loop(0, CW, zrow, 0)
    for k in range(RPT // CW):
        pltpu.async_copy(wr0, shw.at[pl.ds(r0 + k * CW, CW)], gs0)
    for k in range(RPT // CW):
        pltpu.make_async_copy(wr0, shw.at[pl.ds(0, CW)], gs0).wait()
    plsc.subcore_barrier()

    def issue_l(i, k):
        doff = pl.multiple_of((row0 + i) * CW, CW)
        pltpu.async_copy(dst_hbm.at[pl.ds(doff, CW)], dv[k], lsem[k])

    def wait_l(k):
        pltpu.make_async_copy(dst_hbm.at[pl.ds(0, CW)], dv[k], lsem[k]).wait()

    def issue_g(i, k, b):
        woff = pl.multiple_of(jnp.minimum(row0 + i, WROWS - 1) * CW, CW)
        pltpu.async_copy(w_hbm.at[pl.ds(woff, CW)], wr[b], gsem[b])

    def wait_g(k, b):
        pltpu.make_async_copy(w_hbm.at[pl.ds(0, CW)], wr[b], gsem[b]).wait()

    def issue_s(k, b):
        pltpu.async_copy(wr[b], shw.at[dv[k]], ssem[b], add=True)

    def wait_s(k, b):
        pltpu.make_async_copy(wr[b], shw.at[dv[k]], ssem[b]).wait()

    issue_l(0, 0)
    issue_l(1, 1)

    def block(j, carry):
        for k in range(4):
            i = 4 * j + k
            b = k % 2
            wait_l(k)

            @pl.when(i >= 2)
            def _():
                wait_s((k + 2) % 4, b)

            issue_g(i, k, b)

            @pl.when(i >= 1)
            def _():
                wait_g((k + 3) % 4, 1 - b)
                issue_s((k + 3) % 4, 1 - b)

            @pl.when(i + 2 < KW)
            def _():
                issue_l(i + 2, (k + 2) % 4)
        return carry

    lax.fori_loop(0, KW // 4, block, 0)
    wait_g(3, 1)
    issue_s(3, 1)
    wait_s(2, 0)
    wait_s(3, 1)
    plsc.subcore_barrier()
    pltpu.sync_copy(shw.at[pl.ds(r0, RPT)], accw_hbm.at[c, pl.ds(r0, RPT)])


_scw_segsum = functools.partial(
    pl.kernel,
    out_type=jax.ShapeDtypeStruct((NC, N_ACC, DE), jnp.float32),
    mesh=plsc.VectorSubcoreMesh(core_axis_name="c", subcore_axis_name="s"),
    scratch_types=[
        pltpu.VMEM((CW,), jnp.int32),
        pltpu.VMEM((CW,), jnp.int32),
        pltpu.VMEM((CW,), jnp.int32),
        pltpu.VMEM((CW,), jnp.int32),
        pltpu.VMEM((CW, DE), jnp.float32),
        pltpu.VMEM((CW, DE), jnp.float32),
        pltpu.VMEM_SHARED((N_ACC, DE), jnp.float32),
        pltpu.SemaphoreType.DMA,
        pltpu.SemaphoreType.DMA,
        pltpu.SemaphoreType.DMA,
        pltpu.SemaphoreType.DMA,
        pltpu.SemaphoreType.DMA,
        pltpu.SemaphoreType.DMA,
        pltpu.SemaphoreType.DMA,
        pltpu.SemaphoreType.DMA,
    ],
    compiler_params=pltpu.CompilerParams(use_tc_tiling_on_sc=False),
)(_scw_body)


def _prep_body(x_ref, f_ref, ei_ref, o_ref, sd_ref):
    x = x_ref[...]
    q = jnp.sum(x * x, axis=1, keepdims=True)
    one = jnp.ones_like(q)
    pad = jnp.zeros((x.shape[0], PAY - D - 5), jnp.float32)
    o_ref[...] = jnp.concatenate([f_ref[...], x, q, one, pad], axis=1)
    e = ei_ref[...]
    sd_ref[...] = jnp.stack([e[0], e[1]], axis=1).reshape(sd_ref.shape)


def _epi_body(acc_ref, accw_ref, fn_ref, wg_ref, w2_ref, w3_ref, w4_ref,
              wu_ref, bu_ref, o_ref):
    A = acc_ref[0] + acc_ref[1]
    Bw = accw_ref[0] + accw_ref[1]
    fn = fn_ref[...]
    f = fn[:, :D]
    xv = fn[:, D:D + 3]
    q = fn[:, D + 3:D + 4]
    Ax = A[:, D:D + 3]
    Aq = A[:, D + 3:D + 4]
    deg = A[:, D + 4:D + 5]
    s = Aq + deg * q - 2.0 * jnp.sum(xv * Ax, axis=1, keepdims=True)
    m = (jnp.dot(A, wg_ref[...], preferred_element_type=jnp.float32)
         + jnp.dot(Bw, w3_ref[...], preferred_element_type=jnp.float32)
         + deg * jnp.dot(f, w2_ref[...], preferred_element_type=jnp.float32)
         + s * w4_ref[...])
    o_ref[...] = (jnp.dot(m + f, wu_ref[...], preferred_element_type=jnp.float32)
                  + bu_ref[...])


def kernel(x, f, w, W_msg, b_msg, W_upd, b_upd, edge_index):
    x = x.astype(jnp.float32)
    f = f.astype(jnp.float32)
    w = w.astype(jnp.float32)

    ei3 = edge_index.reshape(2, EROWS, C)
    dstw = jnp.concatenate(
        [edge_index[1],
         N + (jnp.arange(WPAD, dtype=jnp.int32) % (N_ACC - N))])

    wg = jnp.zeros((PAY, D), jnp.float32)
    wg = wg.at[:D].set(W_msg[:D])
    wg = wg.at[D + 4].set(b_msg)
    w2 = W_msg[D:2 * D]
    w3 = W_msg[2 * D:2 * D + DE]
    w4 = W_msg[2 * D + DE:2 * D + DE + 1]
    bu = b_upd.reshape(1, D)

    nblk = N // BP
    erb = EROWS // nblk
    fnode, sd = pl.pallas_call(
        _prep_body,
        out_shape=[
            jax.ShapeDtypeStruct((N_ACC, PAY), jnp.float32),
            jax.ShapeDtypeStruct((2 * EROWS, C), jnp.int32),
        ],
        grid=(nblk,),
        in_specs=[
            pl.BlockSpec((BP, 3), lambda i: (i, 0)),
            pl.BlockSpec((BP, D), lambda i: (i, 0)),
            pl.BlockSpec((2, erb, C), lambda i: (0, i, 0)),
        ],
        out_specs=[
            pl.BlockSpec((BP, PAY), lambda i: (i, 0)),
            pl.BlockSpec((2 * erb, C), lambda i: (i, 0)),
        ],
    )(x, f, ei3)

    accf = _sc_segsum(fnode, sd)
    accw = _scw_segsum(w, dstw)

    out = pl.pallas_call(
        _epi_body,
        out_shape=jax.ShapeDtypeStruct((N, D), jnp.float32),
        grid=(N // BN,),
        in_specs=[
            pl.BlockSpec((NC, BN, PAY), lambda i: (0, i, 0)),
            pl.BlockSpec((NC, BN, DE), lambda i: (0, i, 0)),
            pl.BlockSpec((BN, PAY), lambda i: (i, 0)),
            pl.BlockSpec((PAY, D), lambda i: (0, 0)),
            pl.BlockSpec((D, D), lambda i: (0, 0)),
            pl.BlockSpec((DE, D), lambda i: (0, 0)),
            pl.BlockSpec((1, D), lambda i: (0, 0)),
            pl.BlockSpec((D, D), lambda i: (0, 0)),
            pl.BlockSpec((1, D), lambda i: (0, 0)),
        ],
        out_specs=pl.BlockSpec((BN, D), lambda i: (i, 0)),
    )(accf, accw, fnode, wg, w2, w3, w4, W_upd, bu)

    return out


# speedup vs baseline: 14.4413x; 1.0016x over previous
import functools

import jax
import jax.numpy as jnp
from jax import lax
from jax.experimental import pallas as pl
from jax.experimental.pallas import tpu as pltpu
from jax.experimental.pallas import tpu_sc as plsc

N = 10000
E = 320000
D = 128
DE = 16
PAY = 144
NC = 2
NS = 16
NW = NC * NS
C = 80
EROWS = E // C
A0 = 128
EA = 0
B1 = 120
EB = 8
T0 = NS * A0 + 4 * EA
N_ACC = 10240
RPT = N_ACC // NS
BN = 1000
BP = 1000


def _sc_body(fn_hbm, sd_hbm,
             accf_hbm,
             sdv0, sdv1, sdv2, sdv3,
             rows0, rows1, shf,
             ls0, ls1, ls2, ls3,
             gs0, gs1, ss0, ss1):
    c = lax.axis_index("c")
    s = lax.axis_index("s")
    r0 = s * RPT
    nch = jnp.where(c == 0,
                    A0 + 4 * (s < EA).astype(jnp.int32),
                    B1 + 4 * (s < EB).astype(jnp.int32))
    row0 = jnp.where(c == 0,
                     s * A0 + 4 * jnp.minimum(s, EA),
                     T0 + s * B1 + 4 * jnp.minimum(s, EB))

    sdv = (sdv0, sdv1, sdv2, sdv3)
    rows = (rows0, rows1)
    lsem = (ls0, ls1, ls2, ls3)
    gsem = (gs0, gs1)
    ssem = (ss0, ss1)

    zf32 = jnp.zeros((16,), jnp.float32)

    def zrow(i, carry):
        for j in range(PAY // 16):
            rows0[i, pl.ds(j * 16, 16)] = zf32
        return carry

    lax.fori_loop(0, C, zrow, 0)
    for k in range(RPT // C):
        pltpu.async_copy(rows0, shf.at[pl.ds(r0 + k * C, C)], gs0)
    for k in range(RPT // C):
        pltpu.make_async_copy(rows0, shf.at[pl.ds(0, C)], gs0).wait()
    plsc.subcore_barrier()

    def issue_l(i, k):
        soff = pl.multiple_of((row0 + i) * 2, 2)
        pltpu.async_copy(sd_hbm.at[pl.ds(soff, 2)], sdv[k], lsem[k])

    def wait_l(k):
        pltpu.make_async_copy(sd_hbm.at[pl.ds(0, 2)], sdv[k], lsem[k]).wait()

    def issue_g(i, k, b):
        pltpu.async_copy(fn_hbm.at[sdv[k].at[0]], rows[b], gsem[b])

    def wait_g(k, b):
        pltpu.make_async_copy(fn_hbm.at[sdv[k].at[0]], rows[b], gsem[b]).wait()

    def issue_s(k, b):
        pltpu.async_copy(rows[b], shf.at[sdv[k].at[1]], ssem[b], add=True)

    def wait_s(k, b):
        pltpu.make_async_copy(rows[b], shf.at[sdv[k].at[1]], ssem[b]).wait()

    issue_l(0, 0)
    issue_l(1, 1)

    def block(j, carry):
        for k in range(4):
            i = 4 * j + k
            b = k % 2
            wait_l(k)
            @pl.when(i >= 2)
            def _():
                wait_s((k + 2) % 4, b)

            issue_g(i, k, b)

            @pl.when(i >= 1)
            def _():
                wait_g((k + 3) % 4, 1 - b)
                issue_s((k + 3) % 4, 1 - b)

            @pl.when(i + 2 < nch)
            def _():
                issue_l(i + 2, (k + 2) % 4)
        return carry

    lax.fori_loop(0, nch // 4, block, 0)
    wait_g(3, 1)
    issue_s(3, 1)
    wait_s(2, 0)
    wait_s(3, 1)
    plsc.subcore_barrier()
    pltpu.sync_copy(shf.at[pl.ds(r0, RPT)], accf_hbm.at[c, pl.ds(r0, RPT)])


_sc_segsum = functools.partial(
    pl.kernel,
    out_type=jax.ShapeDtypeStruct((NC, N_ACC, PAY), jnp.float32),
    mesh=plsc.VectorSubcoreMesh(core_axis_name="c", subcore_axis_name="s", num_cores=NC, num_subcores=NS),
    scratch_types=[
        pltpu.VMEM((2, C), jnp.int32),
        pltpu.VMEM((2, C), jnp.int32),
        pltpu.VMEM((2, C), jnp.int32),
        pltpu.VMEM((2, C), jnp.int32),
        pltpu.VMEM((C, PAY), jnp.float32),
        pltpu.VMEM((C, PAY), jnp.float32),
        pltpu.VMEM_SHARED((N_ACC, PAY), jnp.float32),
        pltpu.SemaphoreType.DMA,
        pltpu.SemaphoreType.DMA,
        pltpu.SemaphoreType.DMA,
        pltpu.SemaphoreType.DMA,
        pltpu.SemaphoreType.DMA,
        pltpu.SemaphoreType.DMA,
        pltpu.SemaphoreType.DMA,
        pltpu.SemaphoreType.DMA,
    ],
    compiler_params=pltpu.CompilerParams(use_tc_tiling_on_sc=False),
)(_sc_body)


CW = 128
WROWS = E // CW
KW = 80
WPAD = NW * KW * CW - E


def _scw_body(w_hbm, dst_hbm, accw_hbm,
              dv0, dv1, dv2, dv3, wr0, wr1, shw,
              ls0, ls1, ls2, ls3, gs0, gs1, ss0, ss1):
    c = lax.axis_index("c")
    s = lax.axis_index("s")
    r0 = s * RPT
    row0 = (s * NC + c) * KW

    dv = (dv0, dv1, dv2, dv3)
    wr = (wr0, wr1)
    lsem = (ls0, ls1, ls2, ls3)
    gsem = (gs0, gs1)
    ssem = (ss0, ss1)

    zf32 = jnp.zeros((16,), jnp.float32)

    def zrow(i, carry):
        wr0[i, pl.ds(0, 16)] = zf32
        return carry

    lax.fori_loop(0, CW, zrow, 0)
    for k in range(RPT // CW):
        pltpu.async_copy(wr0, shw.at[pl.ds(r0 + k * CW, CW)], gs0)
    for k in range(RPT // CW):
        pltpu.make_async_copy(wr0, shw.at[pl.ds(0, CW)], gs0).wait()
    plsc.subcore_barrier()

    def issue_l(i, k):
        doff = pl.multiple_of((row0 + i) * CW, CW)
        pltpu.async_copy(dst_hbm.at[pl.ds(doff, CW)], dv[k], lsem[k])

    def wait_l(k):
        pltpu.make_async_copy(dst_hbm.at[pl.ds(0, CW)], dv[k], lsem[k]).wait()

    def issue_g(i, k, b):
        woff = pl.multiple_of(jnp.minimum(row0 + i, WROWS - 1) * CW, CW)
        pltpu.async_copy(w_hbm.at[pl.ds(woff, CW)], wr[b], gsem[b])

    def wait_g(k, b):
        pltpu.make_async_copy(w_hbm.at[pl.ds(0, CW)], wr[b], gsem[b]).wait()

    def issue_s(k, b):
        pltpu.async_copy(wr[b], shw.at[dv[k]], ssem[b], add=True)

    def wait_s(k, b):
        pltpu.make_async_copy(wr[b], shw.at[dv[k]], ssem[b]).wait()

    issue_l(0, 0)
    issue_l(1, 1)

    def block(j, carry):
        for k in range(4):
            i = 4 * j + k
            b = k % 2
            wait_l(k)

            @pl.when(i >= 2)
            def _():
                wait_s((k + 2) % 4, b)

            issue_g(i, k, b)

            @pl.when(i >= 1)
            def _():
                wait_g((k + 3) % 4, 1 - b)
                issue_s((k + 3) % 4, 1 - b)

            @pl.when(i + 2 < KW)
            def _():
                issue_l(i + 2, (k + 2) % 4)
        return carry

    lax.fori_loop(0, KW // 4, block, 0)
    wait_g(3, 1)
    issue_s(3, 1)
    wait_s(2, 0)
    wait_s(3, 1)
    plsc.subcore_barrier()
    pltpu.sync_copy(shw.at[pl.ds(r0, RPT)], accw_hbm.at[c, pl.ds(r0, RPT)])


_scw_segsum = functools.partial(
    pl.kernel,
    out_type=jax.ShapeDtypeStruct((NC, N_ACC, DE), jnp.float32),
    mesh=plsc.VectorSubcoreMesh(core_axis_name="c", subcore_axis_name="s", num_cores=NC, num_subcores=NS),
    scratch_types=[
        pltpu.VMEM((CW,), jnp.int32),
        pltpu.VMEM((CW,), jnp.int32),
        pltpu.VMEM((CW,), jnp.int32),
        pltpu.VMEM((CW,), jnp.int32),
        pltpu.VMEM((CW, DE), jnp.float32),
        pltpu.VMEM((CW, DE), jnp.float32),
        pltpu.VMEM_SHARED((N_ACC, DE), jnp.float32),
        pltpu.SemaphoreType.DMA,
        pltpu.SemaphoreType.DMA,
        pltpu.SemaphoreType.DMA,
        pltpu.SemaphoreType.DMA,
        pltpu.SemaphoreType.DMA,
        pltpu.SemaphoreType.DMA,
        pltpu.SemaphoreType.DMA,
        pltpu.SemaphoreType.DMA,
    ],
    compiler_params=pltpu.CompilerParams(use_tc_tiling_on_sc=False),
)(_scw_body)


def _prep_body(x_ref, f_ref, ei_ref, o_ref, sd_ref):
    x = x_ref[...]
    q = jnp.sum(x * x, axis=1, keepdims=True)
    one = jnp.ones_like(q)
    pad = jnp.zeros((x.shape[0], PAY - D - 5), jnp.float32)
    o_ref[...] = jnp.concatenate([f_ref[...], x, q, one, pad], axis=1)
    e = ei_ref[...]
    sd_ref[...] = jnp.stack([e[0], e[1]], axis=1).reshape(sd_ref.shape)


def _epi_body(acc_ref, accw_ref, fn_ref, wg_ref, w2_ref, w3_ref, w4_ref,
              wu_ref, bu_ref, o_ref):
    A = acc_ref[0] + acc_ref[1]
    Bw = accw_ref[0] + accw_ref[1]
    fn = fn_ref[...]
    f = fn[:, :D]
    xv = fn[:, D:D + 3]
    q = fn[:, D + 3:D + 4]
    Ax = A[:, D:D + 3]
    Aq = A[:, D + 3:D + 4]
    deg = A[:, D + 4:D + 5]
    s = Aq + deg * q - 2.0 * jnp.sum(xv * Ax, axis=1, keepdims=True)
    m = (jnp.dot(A, wg_ref[...], preferred_element_type=jnp.float32)
         + jnp.dot(Bw, w3_ref[...], preferred_element_type=jnp.float32)
         + deg * jnp.dot(f, w2_ref[...], preferred_element_type=jnp.float32)
         + s * w4_ref[...])
    o_ref[...] = (jnp.dot(m + f, wu_ref[...], preferred_element_type=jnp.float32)
                  + bu_ref[...])


def kernel(x, f, w, W_msg, b_msg, W_upd, b_upd, edge_index):
    x = x.astype(jnp.float32)
    f = f.astype(jnp.float32)
    w = w.astype(jnp.float32)

    ei3 = edge_index.reshape(2, EROWS, C)
    dstw = jnp.concatenate(
        [edge_index[1],
         N + (jnp.arange(WPAD, dtype=jnp.int32) % (N_ACC - N))])

    wg = jnp.zeros((PAY, D), jnp.float32)
    wg = wg.at[:D].set(W_msg[:D])
    wg = wg.at[D + 4].set(b_msg)
    w2 = W_msg[D:2 * D]
    w3 = W_msg[2 * D:2 * D + DE]
    w4 = W_msg[2 * D + DE:2 * D + DE + 1]
    bu = b_upd.reshape(1, D)

    nblk = N // BP
    erb = EROWS // nblk
    fnode, sd = pl.pallas_call(
        _prep_body,
        out_shape=[
            jax.ShapeDtypeStruct((N_ACC, PAY), jnp.float32),
            jax.ShapeDtypeStruct((2 * EROWS, C), jnp.int32),
        ],
        grid=(nblk,),
        in_specs=[
            pl.BlockSpec((BP, 3), lambda i: (i, 0)),
            pl.BlockSpec((BP, D), lambda i: (i, 0)),
            pl.BlockSpec((2, erb, C), lambda i: (0, i, 0)),
        ],
        out_specs=[
            pl.BlockSpec((BP, PAY), lambda i: (i, 0)),
            pl.BlockSpec((2 * erb, C), lambda i: (i, 0)),
        ],
    )(x, f, ei3)

    accf = _sc_segsum(fnode, sd)
    accw = _scw_segsum(w, dstw)

    out = pl.pallas_call(
        _epi_body,
        out_shape=jax.ShapeDtypeStruct((N, D), jnp.float32),
        grid=(N // BN,),
        in_specs=[
            pl.BlockSpec((NC, BN, PAY), lambda i: (0, i, 0)),
            pl.BlockSpec((NC, BN, DE), lambda i: (0, i, 0)),
            pl.BlockSpec((BN, PAY), lambda i: (i, 0)),
            pl.BlockSpec((PAY, D), lambda i: (0, 0)),
            pl.BlockSpec((D, D), lambda i: (0, 0)),
            pl.BlockSpec((DE, D), lambda i: (0, 0)),
            pl.BlockSpec((1, D), lambda i: (0, 0)),
            pl.BlockSpec((D, D), lambda i: (0, 0)),
            pl.BlockSpec((1, D), lambda i: (0, 0)),
        ],
        out_specs=pl.BlockSpec((BN, D), lambda i: (i, 0)),
    )(accf, accw, fnode, wg, w2, w3, w4, W_upd, bu)

    return out
